# baseline (speedup 1.0000x reference)
import functools

import jax
import jax.numpy as jnp
from jax import lax
from jax.experimental import pallas as pl
from jax.experimental.pallas import tpu as pltpu
from jax.experimental.pallas import tpu_sc as plsc

M = 1000000
D = 32
B = 16384

NC = 2
NS = 16
L = 16
NW = NC * NS

USUP = B // 4
UNIT = 128
BASE_UNITS = 244
WBLK = 512
NBLK = 60
NBUF = 3
NSEL_GROUPS = B // L

CAP = 768
UCHUNK = 128
LCAP = CAP + UCHUNK
IDXH = B // 2
NBKT = 62
BCAP = 48


def _body(dataT, idx_hbm, upd_hbm, outT,
          idx_v, blist, loclist, uq, ubuf0, ubuf1, brow0, brow1,
          cbuf0, cbuf1, cbuf2, tailbuf, bkt, counts,
          sin0, sin1, sin2, sout0, sout1, sout2,
          sg0, sg1):
    wid = lax.axis_index("s") * NC + lax.axis_index("c")
    c_lo = UNIT * (BASE_UNITS * wid + jnp.minimum(wid, 4))
    width = UNIT * (BASE_UNITS + (wid < 4).astype(jnp.int32))
    is_last = wid == NW - 1
    c_hi = c_lo + width + jnp.where(is_last, 64, 0)

    bufs = (cbuf0, cbuf1, cbuf2)
    sins = (sin0, sin1, sin2)
    souts = (sout0, sout1, sout2)
    ubufs = (ubuf0, ubuf1)
    brows = (brow0, brow1)
    sgs = (sg0, sg1)

    def in_slice(k):
        return dataT.at[:, pl.ds(c_lo + k * WBLK, WBLK)]

    def out_slice(k):
        return outT.at[:, pl.ds(c_lo + k * WBLK, WBLK)]

    pltpu.async_copy(in_slice(0), cbuf0, sin0)
    pltpu.async_copy(in_slice(1), cbuf1, sin1)

    lane = lax.iota(jnp.int32, L)
    zeros = jnp.zeros((L,), jnp.int32)

    for g in range(LCAP // L):
        blist[pl.ds(g * L, L)] = zeros

    def make_sel_body(jbase):
        def sel_body(g, off):
            idxv = idx_v[pl.ds(g * L, L)]
            m = (idxv >= c_lo) & (idxv < c_hi)
            cnt = jnp.sum(m.astype(jnp.int32))
            safe = jnp.minimum(off, CAP)
            plsc.store_compressed(blist.at[pl.ds(safe, L)],
                                  jbase + g * L + lane, mask=m)
            plsc.store_compressed(loclist.at[pl.ds(safe, L)], idxv, mask=m)
            return off + cnt
        return sel_body

    pltpu.sync_copy(idx_hbm.at[pl.ds(0, IDXH)], idx_v)
    n_sel = lax.fori_loop(0, IDXH // L, make_sel_body(0), jnp.int32(0))
    pltpu.sync_copy(idx_hbm.at[pl.ds(IDXH, IDXH)], idx_v)
    n_sel = lax.fori_loop(0, IDXH // L, make_sel_body(IDXH), n_sel)
    n_sel = jnp.minimum(n_sel, CAP)
    ngroups = (n_sel + L - 1) // L

    def dedup_body(g, _):
        jv = g * L + lane
        valid = jv < n_sel
        locv = loclist[pl.ds(g * L, L)]
        keep = valid
        for s in range(1, L):
            perm = (lane + s) & (L - 1)
            r = locv.at[perm].get(mode="promise_in_bounds")
            pvalid = (g * L + perm) < n_sel
            keep &= ~((r == locv) & (perm > lane) & pvalid)
        loclist[pl.ds(g * L, L)] = jnp.where(keep, locv, -1)
        return 0

    lax.fori_loop(0, ngroups, dedup_body, 0)

    for g in range(4):
        counts[pl.ds(g * L, L)] = zeros

    def bkt_body(g, _):
        jv = g * L + lane
        valid = jv < n_sel
        locv = loclist[pl.ds(g * L, L)]
        ok = valid & (locv >= 0)
        blk = jnp.where(ok, (locv - c_lo) >> 9, 0)
        for i in range(L):
            mi = ok & (lane == i)
            cnt_i = plsc.load_gather(counts, [blk], mask=mi)
            pos = jnp.minimum(cnt_i, BCAP - 1)
            plsc.store_scatter(bkt, [blk * BCAP + pos], jv, mask=mi)
            plsc.store_scatter(counts, [blk], cnt_i + 1, mask=mi)
        return 0

    lax.fori_loop(0, ngroups, bkt_body, 0)

    nuch = (n_sel + UCHUNK - 1) // UCHUNK

    def load_chunk(p, c):
        base = c * UCHUNK
        for g in range(UCHUNK // L):
            brows[p][pl.ds(g * L, L)] = blist[pl.ds(base + g * L, L)] >> 2
        pltpu.async_copy(upd_hbm.at[brows[p]], ubufs[p], sgs[p])

    def extract_chunk(p, c):
        base = c * UCHUNK
        pltpu.make_async_copy(upd_hbm.at[brows[p]], ubufs[p], sgs[p]).wait()
        for g in range(UCHUNK // L):
            jv = base + g * L + lane
            bv = blist[pl.ds(base + g * L, L)]
            qcol = (bv & 3) * D
            srow = jnp.full((L,), g * L, jnp.int32) + lane
            for e in range(D):
                val = plsc.load_gather(ubufs[p], [srow, qcol + e])
                plsc.store_scatter(uq, [jv >> 2, (jv & 3) * D + e], val)

    load_chunk(0, 0)

    @pl.when(nuch > 1)
    def _g1():
        load_chunk(1, 1)

    def uq_pair(kk, _):
        for p in range(2):
            c = 2 * kk + p

            @pl.when(c < nuch)
            def _do():
                extract_chunk(p, c)

                @pl.when(c + 2 < nuch)
                def _pf():
                    load_chunk(p, c + 2)
        return 0

    lax.fori_loop(0, (nuch + 1) // 2, uq_pair, 0)

    def _patch(buf, cblk, blksz, bid):
        bidv = jnp.full((L,), bid, jnp.int32)
        nkv = plsc.load_gather(counts, [bidv])
        for g in range(BCAP // L):
            ev = g * L + lane
            valid = ev < nkv
            cnt = jnp.sum(valid.astype(jnp.int32))

            @pl.when(cnt > 0)
            def _do():
                jv = plsc.load_gather(bkt, [bidv * BCAP + ev], mask=valid)
                jv = jnp.where(valid, jv, 0)
                locv = plsc.load_gather(loclist, [jv], mask=valid)
                inblk = valid & (locv >= cblk) & (locv < cblk + blksz)
                cloc = jnp.where(inblk, locv - cblk, 0)
                urow = jv >> 2
                ucol = (jv & 3) * D
                for e in range(D):
                    val = plsc.load_gather(uq, [urow, ucol + e], mask=inblk)
                    plsc.store_scatter(buf, [jnp.full((L,), e, jnp.int32),
                                             cloc], val, mask=inblk)

    def quad(kk, _):
        for p in range(NBUF):
            k = NBUF * kk + p
            pltpu.make_async_copy(in_slice(k), bufs[p], sins[p]).wait()
            _patch(bufs[p], c_lo + k * WBLK, WBLK, k)
            pltpu.async_copy(bufs[p], out_slice(k), souts[p])

            q = (p + 2) % NBUF

            @pl.when((k >= NBUF - 2) & (k + 2 < NBLK))
            def _wprev():
                pltpu.make_async_copy(bufs[q], out_slice(k - 2),
                                      souts[q]).wait()

            @pl.when(k + 2 < NBLK)
            def _pf():
                pltpu.async_copy(in_slice(k + 2), bufs[q], sins[q])
        return 0

    lax.fori_loop(0, NBLK // NBUF, quad, 0)

    for j in range(NBLK - NBUF, NBLK):
        pltpu.make_async_copy(bufs[j % NBUF], out_slice(j),
                              souts[j % NBUF]).wait()

    ntail = (width - NBLK * WBLK) // UNIT
    tslice = cbuf0.at[:, pl.ds(0, UNIT)]

    def tail_body(t, _):
        base = c_lo + NBLK * WBLK + t * UNIT
        pltpu.sync_copy(dataT.at[:, pl.ds(base, UNIT)], tslice)
        _patch(cbuf0, base, UNIT, (NBLK * WBLK + t * UNIT) >> 9)
        pltpu.sync_copy(tslice, outT.at[:, pl.ds(base, UNIT)])
        return 0

    lax.fori_loop(0, ntail, tail_body, 0)

    @pl.when(is_last)
    def _ragged():
        base = (M // UNIT) * UNIT
        rag = M - (M // UNIT) * UNIT
        pltpu.sync_copy(dataT.at[:, pl.ds(base, rag)], tailbuf)
        _patch(tailbuf, base, rag, NBKT - 1)
        pltpu.sync_copy(tailbuf, outT.at[:, pl.ds(base, rag)])


@functools.partial(
    pl.kernel,
    out_type=jax.ShapeDtypeStruct((D, M), jnp.float32),
    mesh=plsc.VectorSubcoreMesh(
        core_axis_name="c", subcore_axis_name="s", num_cores=NC,
        num_subcores=NS),
    scratch_types=[
        pltpu.VMEM((IDXH,), jnp.int32),
        pltpu.VMEM((LCAP,), jnp.int32),
        pltpu.VMEM((LCAP,), jnp.int32),
        pltpu.VMEM((LCAP // 4, 4 * D), jnp.float32),
        pltpu.VMEM((UCHUNK, 4 * D), jnp.float32),
        pltpu.VMEM((UCHUNK, 4 * D), jnp.float32),
        pltpu.VMEM((UCHUNK,), jnp.int32),
        pltpu.VMEM((UCHUNK,), jnp.int32),
        pltpu.VMEM((D, WBLK), jnp.float32),
        pltpu.VMEM((D, WBLK), jnp.float32),
        pltpu.VMEM((D, WBLK), jnp.float32),
        pltpu.VMEM((D, M % UNIT), jnp.float32),
        pltpu.VMEM((NBKT * BCAP,), jnp.int32),
        pltpu.VMEM((4 * L,), jnp.int32),
        pltpu.SemaphoreType.DMA,
        pltpu.SemaphoreType.DMA,
        pltpu.SemaphoreType.DMA,
        pltpu.SemaphoreType.DMA,
        pltpu.SemaphoreType.DMA,
        pltpu.SemaphoreType.DMA,
        pltpu.SemaphoreType.DMA,
        pltpu.SemaphoreType.DMA,
    ],
    compiler_params=pltpu.CompilerParams(needs_layout_passes=False),
)
def _scatter_nd_sc(dataT, idx_hbm, upd_hbm, outT, *scratch):
    _body(dataT, idx_hbm, upd_hbm, outT, *scratch)


def kernel(data, indices, updates):
    dataT = data.T
    upd4 = updates.reshape(USUP, 4 * D)
    outT = _scatter_nd_sc(dataT, indices.reshape(B), upd4)
    return outT.T


# speedup vs baseline: 1.2880x; 1.2880x over previous
import functools

import jax
import jax.numpy as jnp
from jax import lax
from jax.experimental import pallas as pl
from jax.experimental.pallas import tpu as pltpu
from jax.experimental.pallas import tpu_sc as plsc

M = 1000000
D = 32
B = 16384

NC = 2
NS = 16
L = 16
NW = NC * NS

USUP = B // 4
UNIT = 128
BASE_UNITS = 244
WBLK = 512
NBLK = 60
NBUF = 4
NSEL_GROUPS = B // L

CAP = 768
UCHUNK = 64
LCAP = CAP + UCHUNK
IDXH = B // 2
NBKT = 62
BCAP = 48


def _body(dataT, idx_hbm, upd_hbm, outT,
          idx_v, blist, loclist, uq, ubuf0, ubuf1, brow0, brow1,
          cbuf0, cbuf1, cbuf2, cbuf3, tailbuf, bkt, counts,
          sin0, sin1, sin2, sin3, sout0, sout1, sout2, sout3,
          sg0, sg1):
    wid = lax.axis_index("s") * NC + lax.axis_index("c")
    c_lo = UNIT * (BASE_UNITS * wid + jnp.minimum(wid, 4))
    width = UNIT * (BASE_UNITS + (wid < 4).astype(jnp.int32))
    is_last = wid == NW - 1
    c_hi = c_lo + width + jnp.where(is_last, 64, 0)

    bufs = (cbuf0, cbuf1, cbuf2, cbuf3)
    sins = (sin0, sin1, sin2, sin3)
    souts = (sout0, sout1, sout2, sout3)
    ubufs = (ubuf0, ubuf1)
    brows = (brow0, brow1)
    sgs = (sg0, sg1)

    def in_slice(k):
        return dataT.at[:, pl.ds(c_lo + k * WBLK, WBLK)]

    def out_slice(k):
        return outT.at[:, pl.ds(c_lo + k * WBLK, WBLK)]

    pltpu.async_copy(in_slice(0), cbuf0, sin0)
    pltpu.async_copy(in_slice(1), cbuf1, sin1)

    lane = lax.iota(jnp.int32, L)
    zeros = jnp.zeros((L,), jnp.int32)

    for g in range(LCAP // L):
        blist[pl.ds(g * L, L)] = zeros

    def make_sel_body(jbase):
        def sel_body(g, off):
            idxv = idx_v[pl.ds(g * L, L)]
            m = (idxv >= c_lo) & (idxv < c_hi)
            cnt = jnp.sum(m.astype(jnp.int32))
            safe = jnp.minimum(off, CAP)
            plsc.store_compressed(blist.at[pl.ds(safe, L)],
                                  jbase + g * L + lane, mask=m)
            plsc.store_compressed(loclist.at[pl.ds(safe, L)], idxv, mask=m)
            return off + cnt
        return sel_body

    pltpu.sync_copy(idx_hbm.at[pl.ds(0, IDXH)], idx_v)
    n_sel = lax.fori_loop(0, IDXH // L, make_sel_body(0), jnp.int32(0))
    pltpu.sync_copy(idx_hbm.at[pl.ds(IDXH, IDXH)], idx_v)
    n_sel = lax.fori_loop(0, IDXH // L, make_sel_body(IDXH), n_sel)
    n_sel = jnp.minimum(n_sel, CAP)
    ngroups = (n_sel + L - 1) // L

    def dedup_body(g, _):
        jv = g * L + lane
        valid = jv < n_sel
        locv = loclist[pl.ds(g * L, L)]
        keep = valid
        for s in range(1, L):
            perm = (lane + s) & (L - 1)
            r = locv.at[perm].get(mode="promise_in_bounds")
            pvalid = (g * L + perm) < n_sel
            keep &= ~((r == locv) & (perm > lane) & pvalid)
        loclist[pl.ds(g * L, L)] = jnp.where(keep, locv, -1)
        return 0

    lax.fori_loop(0, ngroups, dedup_body, 0)

    for g in range(4):
        counts[pl.ds(g * L, L)] = zeros

    def bkt_body(g, _):
        jv = g * L + lane
        valid = jv < n_sel
        locv = loclist[pl.ds(g * L, L)]
        ok = valid & (locv >= 0)
        blk = jnp.where(ok, (locv - c_lo) >> 9, 0)
        for i in range(L):
            mi = ok & (lane == i)
            cnt_i = plsc.load_gather(counts, [blk], mask=mi)
            pos = jnp.minimum(cnt_i, BCAP - 1)
            plsc.store_scatter(bkt, [blk * BCAP + pos], jv, mask=mi)
            plsc.store_scatter(counts, [blk], cnt_i + 1, mask=mi)
        return 0

    lax.fori_loop(0, ngroups, bkt_body, 0)

    nuch = (n_sel + UCHUNK - 1) // UCHUNK

    def load_chunk(p, c):
        base = c * UCHUNK
        for g in range(UCHUNK // L):
            brows[p][pl.ds(g * L, L)] = blist[pl.ds(base + g * L, L)] >> 2
        pltpu.async_copy(upd_hbm.at[brows[p]], ubufs[p], sgs[p])

    def extract_chunk(p, c):
        base = c * UCHUNK
        pltpu.make_async_copy(upd_hbm.at[brows[p]], ubufs[p], sgs[p]).wait()
        for g in range(UCHUNK // L):
            jv = base + g * L + lane
            bv = blist[pl.ds(base + g * L, L)]
            qcol = (bv & 3) * D
            srow = jnp.full((L,), g * L, jnp.int32) + lane
            for e in range(D):
                val = plsc.load_gather(ubufs[p], [srow, qcol + e])
                plsc.store_scatter(uq, [jv >> 2, (jv & 3) * D + e], val)

    load_chunk(0, 0)

    @pl.when(nuch > 1)
    def _g1():
        load_chunk(1, 1)

    def uq_pair(kk, _):
        for p in range(2):
            c = 2 * kk + p

            @pl.when(c < nuch)
            def _do():
                extract_chunk(p, c)

                @pl.when(c + 2 < nuch)
                def _pf():
                    load_chunk(p, c + 2)
        return 0

    lax.fori_loop(0, (nuch + 1) // 2, uq_pair, 0)

    def _patch(buf, cblk, blksz, bid):
        bidv = jnp.full((L,), bid, jnp.int32)
        nkv = plsc.load_gather(counts, [bidv])
        for g in range(BCAP // L):
            ev = g * L + lane
            valid = ev < nkv
            cnt = jnp.sum(valid.astype(jnp.int32))

            @pl.when(cnt > 0)
            def _do():
                jv = plsc.load_gather(bkt, [bidv * BCAP + ev], mask=valid)
                jv = jnp.where(valid, jv, 0)
                locv = plsc.load_gather(loclist, [jv], mask=valid)
                inblk = valid & (locv >= cblk) & (locv < cblk + blksz)
                cloc = jnp.where(inblk, locv - cblk, 0)
                urow = jv >> 2
                ucol = (jv & 3) * D
                for e in range(D):
                    val = plsc.load_gather(uq, [urow, ucol + e], mask=inblk)
                    plsc.store_scatter(buf, [jnp.full((L,), e, jnp.int32),
                                             cloc], val, mask=inblk)

    def quad(kk, _):
        for p in range(NBUF):
            k = NBUF * kk + p
            pltpu.make_async_copy(in_slice(k), bufs[p], sins[p]).wait()
            _patch(bufs[p], c_lo + k * WBLK, WBLK, k)
            pltpu.async_copy(bufs[p], out_slice(k), souts[p])

            q = (p + 2) % NBUF

            @pl.when((k >= 2) & (k + 2 < NBLK))
            def _wprev():
                pltpu.make_async_copy(bufs[q], out_slice(k - 2),
                                      souts[q]).wait()

            @pl.when(k + 2 < NBLK)
            def _pf():
                pltpu.async_copy(in_slice(k + 2), bufs[q], sins[q])
        return 0

    lax.fori_loop(0, NBLK // NBUF, quad, 0)

    for q in range(NBUF):
        pltpu.make_async_copy(bufs[q], out_slice(NBLK - NBUF + q),
                              souts[q]).wait()

    ntail = (width - NBLK * WBLK) // UNIT
    tslice = cbuf0.at[:, pl.ds(0, UNIT)]

    def tail_body(t, _):
        base = c_lo + NBLK * WBLK + t * UNIT
        pltpu.sync_copy(dataT.at[:, pl.ds(base, UNIT)], tslice)
        _patch(cbuf0, base, UNIT, (NBLK * WBLK + t * UNIT) >> 9)
        pltpu.sync_copy(tslice, outT.at[:, pl.ds(base, UNIT)])
        return 0

    lax.fori_loop(0, ntail, tail_body, 0)

    @pl.when(is_last)
    def _ragged():
        base = (M // UNIT) * UNIT
        rag = M - (M // UNIT) * UNIT
        pltpu.sync_copy(dataT.at[:, pl.ds(base, rag)], tailbuf)
        _patch(tailbuf, base, rag, NBKT - 1)
        pltpu.sync_copy(tailbuf, outT.at[:, pl.ds(base, rag)])


@functools.partial(
    pl.kernel,
    out_type=jax.ShapeDtypeStruct((D, M), jnp.float32),
    mesh=plsc.VectorSubcoreMesh(
        core_axis_name="c", subcore_axis_name="s", num_cores=NC,
        num_subcores=NS),
    scratch_types=[
        pltpu.VMEM((IDXH,), jnp.int32),
        pltpu.VMEM((LCAP,), jnp.int32),
        pltpu.VMEM((LCAP,), jnp.int32),
        pltpu.VMEM((LCAP // 4, 4 * D), jnp.float32),
        pltpu.VMEM((UCHUNK, 4 * D), jnp.float32),
        pltpu.VMEM((UCHUNK, 4 * D), jnp.float32),
        pltpu.VMEM((UCHUNK,), jnp.int32),
        pltpu.VMEM((UCHUNK,), jnp.int32),
        pltpu.VMEM((D, WBLK), jnp.float32),
        pltpu.VMEM((D, WBLK), jnp.float32),
        pltpu.VMEM((D, WBLK), jnp.float32),
        pltpu.VMEM((D, WBLK), jnp.float32),
        pltpu.VMEM((D, M % UNIT), jnp.float32),
        pltpu.VMEM((NBKT * BCAP,), jnp.int32),
        pltpu.VMEM((4 * L,), jnp.int32),
        pltpu.SemaphoreType.DMA,
        pltpu.SemaphoreType.DMA,
        pltpu.SemaphoreType.DMA,
        pltpu.SemaphoreType.DMA,
        pltpu.SemaphoreType.DMA,
        pltpu.SemaphoreType.DMA,
        pltpu.SemaphoreType.DMA,
        pltpu.SemaphoreType.DMA,
        pltpu.SemaphoreType.DMA,
        pltpu.SemaphoreType.DMA,
    ],
    compiler_params=pltpu.CompilerParams(needs_layout_passes=False),
)
def _scatter_nd_sc(dataT, idx_hbm, upd_hbm, outT, *scratch):
    _body(dataT, idx_hbm, upd_hbm, outT, *scratch)


def kernel(data, indices, updates):
    dataT = data.T
    upd4 = updates.reshape(USUP, 4 * D)
    outT = _scatter_nd_sc(dataT, indices.reshape(B), upd4)
    return outT.T
